# baseline (speedup 1.0000x reference)
import jax
import jax.numpy as jnp
from jax import lax
from jax.experimental import pallas as pl
from jax.experimental.pallas import tpu as pltpu
from jax.experimental.pallas import tpu_sc as plsc

_SHIFT_FACTOR = 2
_B, _C, _T = 32, 128, 16384
_L = 16
_NCHUNK = _T // _L
_NIN = 2
_NOUT = 4


def _make_body(nc):
  def _body(x_hbm, sh_hbm, out_hbm, svec, xb0, xb1, ob0, ob1, ob2, ob3,
            sin0, sin1, sout0, sout1, sout2, sout3):
    wid = lax.axis_index("s") * nc + lax.axis_index("c")
    xbufs, obufs = (xb0, xb1), (ob0, ob1, ob2, ob3)
    sins, souts = (sin0, sin1), (sout0, sout1, sout2, sout3)

    pltpu.sync_copy(sh_hbm.at[wid], svec)
    s = svec[...][0]

    def shift_idx(t):
        pos = jnp.where(t >= s, t - s, t)
        neg = jnp.minimum(t - s, _T - 1)
        return jnp.where(s > 0, pos, jnp.where(s < 0, neg, t))

    t0 = lax.iota(jnp.int32, 16)
    idx_head = shift_idx(t0)
    idx_tail = shift_idx(t0 + (_T - _L))

    def compute(xrow, orow):
        orow[pl.ds(0, _L)] = plsc.load_gather(xrow, [idx_head])
        orow[pl.ds(_T - _L, _L)] = plsc.load_gather(xrow, [idx_tail])

        @plsc.parallel_loop(1, _NCHUNK - 1, unroll=8)
        def _(i):
            orow[pl.ds(i * _L, _L)] = xrow[pl.ds(i * _L - s, _L)]

    pltpu.make_async_copy(x_hbm.at[wid, 0], xbufs[0], sins[0]).start()

    _STEP = 4
    @pl.loop(0, _C, step=_STEP)
    def _(c):
        for b in range(_STEP):
            r = c + b
            bi, bo = b % _NIN, b

            @pl.when(r + 1 < _C)
            def _():
                pltpu.make_async_copy(
                    x_hbm.at[wid, r + 1], xbufs[1 - bi], sins[1 - bi]
                ).start()

            pltpu.make_async_copy(x_hbm.at[wid, r], xbufs[bi], sins[bi]).wait()

            @pl.when(r >= _NOUT)
            def _():
                pltpu.make_async_copy(
                    obufs[bo], out_hbm.at[wid, r - _NOUT], souts[bo]
                ).wait()

            compute(xbufs[bi], obufs[bo])

            @pl.when(r >= 1)
            def _():
                pltpu.make_async_copy(
                    obufs[(b - 1) % _NOUT],
                    out_hbm.at[wid, r - 1],
                    souts[(b - 1) % _NOUT],
                ).start()

    for k in range(_NOUT - 1):
        pltpu.make_async_copy(
            obufs[k], out_hbm.at[wid, _C - _NOUT + k], souts[k]
        ).wait()
    pltpu.make_async_copy(
        obufs[(_C - 1) % _NOUT], out_hbm.at[wid, _C - 1],
        souts[(_C - 1) % _NOUT],
    ).start()
    pltpu.make_async_copy(
        obufs[(_C - 1) % _NOUT], out_hbm.at[wid, _C - 1],
        souts[(_C - 1) % _NOUT],
    ).wait()

  return _body


def kernel(x):
    B, C, T = x.shape
    skey = jax.random.key(42)
    shifts = jax.random.randint(skey, (B,), -_SHIFT_FACTOR, _SHIFT_FACTOR + 1)
    shifts16 = jnp.broadcast_to(
        shifts.astype(jnp.int32)[:, None], (B, _L)
    )

    mesh = plsc.VectorSubcoreMesh(
        core_axis_name="c", subcore_axis_name="s", num_cores=2, num_subcores=16
    )
    run = pl.kernel(
        _make_body(mesh.num_cores),
        out_type=jax.ShapeDtypeStruct((B, C, T), jnp.float32),
        mesh=mesh,
        scratch_types=[
            pltpu.VMEM((_L,), jnp.int32),
            pltpu.VMEM((_T,), jnp.float32),
            pltpu.VMEM((_T,), jnp.float32),
            pltpu.VMEM((_T,), jnp.float32),
            pltpu.VMEM((_T,), jnp.float32),
            pltpu.VMEM((_T,), jnp.float32),
            pltpu.VMEM((_T,), jnp.float32),
            pltpu.SemaphoreType.DMA,
            pltpu.SemaphoreType.DMA,
            pltpu.SemaphoreType.DMA,
            pltpu.SemaphoreType.DMA,
            pltpu.SemaphoreType.DMA,
            pltpu.SemaphoreType.DMA,
        ],
        compiler_params=pltpu.CompilerParams(needs_layout_passes=False),
    )
    return run(x, shifts16)
